# baseline (speedup 1.0000x reference)
import functools

import jax
import jax.numpy as jnp
from jax import lax
from jax.experimental import pallas as pl
from jax.experimental.pallas import tpu as pltpu
from jax.experimental.pallas import tpu_sc as plsc

_N = 20000
_K = 6000
_NOUT = 300
_IOU = 0.7
_SCALE = 1000.0
_ROWS = 160
_LANES = 128
_P = _ROWS * _LANES
_C = 6144
_CROWS = 48
_OUTLEN = _C + (_P - _K)
_NEG = -1e9

_NWORKERS = 32
_PER_W = _P // _NWORKERS
_CHUNK = 128


def _prep_body(c0_ref, c1_ref, c2_ref, c3_ref, s_ref,
               ox1, oy1, ox2, oy2, opos):
    f32 = jnp.float32
    i32 = jnp.int32
    imin = jnp.int32(-2147483648)

    row_i = lax.broadcasted_iota(i32, (_ROWS, _LANES), 0)
    lane_i = lax.broadcasted_iota(i32, (_ROWS, _LANES), 1)
    flat_i = row_i * _LANES + lane_i
    valid = flat_i < _N

    b0 = c0_ref[:] * _SCALE
    b1 = c1_ref[:] * _SCALE
    b2 = c2_ref[:] * _SCALE
    b3 = c3_ref[:] * _SCALE
    x1 = jnp.minimum(b0, b2)
    x2 = jnp.maximum(b0, b2)
    y1 = jnp.minimum(b1, b3)
    y2 = jnp.maximum(b1, b3)
    scores = s_ref[:]

    bits = lax.bitcast_convert_type(scores, i32)
    akey = bits ^ (lax.shift_right_arithmetic(bits, 31) & jnp.int32(0x7FFFFFFF))
    akey = jnp.where(valid, akey, imin)
    inv = _P - flat_i

    Tf = imin
    Ti = jnp.int32(0)
    for b in range(31, -1, -1):
        trial = (Tf ^ imin) if b == 31 else (Tf | jnp.int32(1 << b))
        cnt = jnp.sum((akey >= trial).astype(i32))
        Tf = jnp.where(cnt >= _K, trial, Tf)
    for b in range(14, -1, -1):
        trial = Ti | jnp.int32(1 << b)
        cond = (akey > Tf) | ((akey == Tf) & (inv >= trial))
        cnt = jnp.sum(cond.astype(i32))
        Ti = jnp.where(cnt >= _K, trial, Ti)
    in_set = (akey > Tf) | ((akey == Tf) & (inv >= Ti))

    bf16 = jnp.bfloat16
    mask_bf = in_set.astype(bf16)
    up = (lax.broadcasted_iota(i32, (_LANES, _LANES), 0)
          < lax.broadcasted_iota(i32, (_LANES, _LANES), 1)).astype(bf16)
    lane_excl = lax.dot_general(mask_bf, up, (((1,), (0,)), ((), ())),
                                preferred_element_type=jnp.float32)
    rowsum = jnp.sum(in_set.astype(f32), axis=1, keepdims=True)
    lo = (lax.broadcasted_iota(i32, (_ROWS, _ROWS), 0)
          > lax.broadcasted_iota(i32, (_ROWS, _ROWS), 1)).astype(bf16)
    row_excl = lax.dot_general(lo, rowsum.astype(bf16), (((1,), (0,)), ((), ())),
                               preferred_element_type=jnp.float32)
    rank = (row_excl + lane_excl).astype(i32)
    pos = jnp.where(in_set, rank, _C + flat_i - rank)

    ox1[:] = x1
    oy1[:] = y1
    ox2[:] = x2
    oy2[:] = y2
    opos[:] = pos


def _sc_scatter_body(pos_hbm, x1_hbm, y1_hbm, x2_hbm, y2_hbm, s_hbm,
                     o1, o2, o3, o4, o5, idx_v, val_v, sem_in, sem_out):
    wid = lax.axis_index("s") * 2 + lax.axis_index("c")
    base = wid * _PER_W
    comps_in = (x1_hbm, y1_hbm, x2_hbm, y2_hbm, s_hbm)
    comps_out = (o1, o2, o3, o4, o5)

    @pl.loop(0, _PER_W, step=_CHUNK)
    def _(off):
        start = base + off
        sl = pl.ds(start, _CHUNK)
        loads = [pltpu.async_copy(pos_hbm.at[sl], idx_v, sem_in)]
        for c in range(5):
            loads.append(
                pltpu.async_copy(comps_in[c].at[sl], val_v.at[c], sem_in))
        for h in loads:
            h.wait()
        stores = []
        for c in range(5):
            stores.append(
                pltpu.async_copy(val_v.at[c], comps_out[c].at[idx_v], sem_out))
        for h in stores:
            h.wait()


def _nms_body(x1_ref, y1_ref, x2_ref, y2_ref, s_ref, out_ref):
    f32 = jnp.float32
    i32 = jnp.int32

    row_i = lax.broadcasted_iota(i32, (_CROWS, _LANES), 0)
    lane_i = lax.broadcasted_iota(i32, (_CROWS, _LANES), 1)
    flat_i = row_i * _LANES + lane_i
    lane1 = lax.broadcasted_iota(i32, (1, _LANES), 1)

    x1 = x1_ref[:]
    y1 = y1_ref[:]
    x2 = x2_ref[:]
    y2 = y2_ref[:]
    s0 = jnp.where(flat_i < _K, s_ref[:], f32(_NEG))
    areas = (x2 - x1) * (y2 - y1)
    neg_inf = f32(-jnp.inf)

    def pick(ref, row, lmask):
        v = ref[pl.ds(row, 1), :]
        return jnp.max(jnp.where(lmask, v, neg_inf))

    def step(i, carry):
        s, dx1, dy1, dx2, dy2, ds = carry
        m = jnp.max(s)
        idx = jnp.min(jnp.where(s == m, flat_i, _C))
        mask2 = flat_i == idx
        row = idx // _LANES
        lmask = lane1 == (idx - row * _LANES)
        sx1 = pick(x1_ref, row, lmask)
        sy1 = pick(y1_ref, row, lmask)
        sx2 = pick(x2_ref, row, lmask)
        sy2 = pick(y2_ref, row, lmask)
        ssc = pick(s_ref, row, lmask)

        is_deg = m == f32(_NEG)
        dx1 = jnp.where(i == 0, sx1, dx1)
        dy1 = jnp.where(i == 0, sy1, dy1)
        dx2 = jnp.where(i == 0, sx2, dx2)
        dy2 = jnp.where(i == 0, sy2, dy2)
        ds = jnp.where(i == 0, ssc, ds)
        sx1 = jnp.where(is_deg, dx1, sx1)
        sy1 = jnp.where(is_deg, dy1, sy1)
        sx2 = jnp.where(is_deg, dx2, sx2)
        sy2 = jnp.where(is_deg, dy2, sy2)
        ssc = jnp.where(is_deg, ds, ssc)

        xx1 = jnp.maximum(sx1, x1)
        yy1 = jnp.maximum(sy1, y1)
        xx2 = jnp.minimum(sx2, x2)
        yy2 = jnp.minimum(sy2, y2)
        w = jnp.maximum(xx2 - xx1, f32(0.0))
        h = jnp.maximum(yy2 - yy1, f32(0.0))
        inter = w * h
        sel_area = (sx2 - sx1) * (sy2 - sy1)
        iou = inter / (areas + sel_area - inter + f32(1e-9))
        s = jnp.where((iou > f32(_IOU)) | mask2, f32(_NEG), s)

        out = (jnp.where(lane1 == 0, sx1, f32(0.0))
               + jnp.where(lane1 == 1, sy1, f32(0.0))
               + jnp.where(lane1 == 2, sx2, f32(0.0))
               + jnp.where(lane1 == 3, sy2, f32(0.0))
               + jnp.where(lane1 == 4, ssc, f32(0.0)))
        out_ref[pl.ds(i, 1), :] = out
        return (s, dx1, dy1, dx2, dy2, ds)

    zero = f32(0.0)
    lax.fori_loop(0, _NOUT, step, (s0, zero, zero, zero, zero, zero))


@jax.jit
def kernel(boxes, scores):
    pad = _P - _N
    comps = [
        jnp.pad(boxes[:, i], (0, pad)).reshape(_ROWS, _LANES) for i in range(4)
    ]
    s = jnp.pad(scores, (0, pad)).reshape(_ROWS, _LANES)

    shp = jax.ShapeDtypeStruct((_ROWS, _LANES), jnp.float32)
    x1, y1, x2, y2, pos = pl.pallas_call(
        _prep_body,
        out_shape=[shp, shp, shp, shp,
                   jax.ShapeDtypeStruct((_ROWS, _LANES), jnp.int32)],
    )(*comps, s)

    mesh = plsc.VectorSubcoreMesh(core_axis_name="c", subcore_axis_name="s")
    flat_f32 = jax.ShapeDtypeStruct((_OUTLEN,), jnp.float32)
    sc_scatter = pl.kernel(
        _sc_scatter_body,
        out_type=[flat_f32] * 5,
        mesh=mesh,
        scratch_types=[
            pltpu.VMEM((_CHUNK,), jnp.int32),
            pltpu.VMEM((5, _CHUNK), jnp.float32),
            pltpu.SemaphoreType.DMA,
            pltpu.SemaphoreType.DMA,
        ],
    )
    cx1, cy1, cx2, cy2, cs = sc_scatter(
        pos.reshape(_P), x1.reshape(_P), y1.reshape(_P),
        x2.reshape(_P), y2.reshape(_P), s.reshape(_P))

    out = pl.pallas_call(
        _nms_body,
        out_shape=jax.ShapeDtypeStruct((_NOUT, _LANES), jnp.float32),
    )(cx1[:_C].reshape(_CROWS, _LANES), cy1[:_C].reshape(_CROWS, _LANES),
      cx2[:_C].reshape(_CROWS, _LANES), cy2[:_C].reshape(_CROWS, _LANES),
      cs[:_C].reshape(_CROWS, _LANES))
    return out[:, :5]


# speedup vs baseline: 2.5979x; 2.5979x over previous
import functools

import jax
import jax.numpy as jnp
from jax import lax
from jax.experimental import pallas as pl
from jax.experimental.pallas import tpu as pltpu

_N = 20000
_K = 6000
_NOUT = 300
_IOU = 0.7
_SCALE = 1000.0
_ROWS = 160
_LANES = 128
_P = _ROWS * _LANES
_NEG = -1e9


def _nms_body(c0_ref, c1_ref, c2_ref, c3_ref, s_ref, out_ref,
              x1s, y1s, x2s, y2s, areas_s):
    f32 = jnp.float32
    i32 = jnp.int32
    imin = jnp.int32(-2147483648)

    row_i = lax.broadcasted_iota(i32, (_ROWS, _LANES), 0)
    lane_i = lax.broadcasted_iota(i32, (_ROWS, _LANES), 1)
    flat_i = row_i * _LANES + lane_i
    lane1 = lax.broadcasted_iota(i32, (1, _LANES), 1)
    valid = flat_i < _N

    b0 = c0_ref[:] * _SCALE
    b1 = c1_ref[:] * _SCALE
    b2 = c2_ref[:] * _SCALE
    b3 = c3_ref[:] * _SCALE
    x1 = jnp.minimum(b0, b2)
    x2 = jnp.maximum(b0, b2)
    y1 = jnp.minimum(b1, b3)
    y2 = jnp.maximum(b1, b3)
    scores = s_ref[:]
    x1s[:] = x1
    y1s[:] = y1
    x2s[:] = x2
    y2s[:] = y2
    areas_s[:] = (x2 - x1) * (y2 - y1)

    bits = lax.bitcast_convert_type(scores, i32)
    akey = bits ^ (lax.shift_right_arithmetic(bits, 31) & jnp.int32(0x7FFFFFFF))
    akey = jnp.where(valid, akey, imin)
    inv = _P - flat_i

    Tf = imin
    Ti = jnp.int32(0)
    for b in range(31, -1, -1):
        trial = (Tf ^ imin) if b == 31 else (Tf | jnp.int32(1 << b))
        cnt = jnp.sum((akey >= trial).astype(i32))
        Tf = jnp.where(cnt >= _K, trial, Tf)
    for b in range(14, -1, -1):
        trial = Ti | jnp.int32(1 << b)
        cond = (akey > Tf) | ((akey == Tf) & (inv >= trial))
        cnt = jnp.sum(cond.astype(i32))
        Ti = jnp.where(cnt >= _K, trial, Ti)
    in_set = (akey > Tf) | ((akey == Tf) & (inv >= Ti))

    s0 = jnp.where(in_set, scores, f32(_NEG))
    neg_inf = f32(-jnp.inf)

    def step(i, carry):
        s, dx1, dy1, dx2, dy2, ds = carry
        m = jnp.max(s)
        idx = jnp.min(jnp.where(s == m, flat_i, _P))
        mask2 = flat_i == idx
        row = idx // _LANES
        lmask = lane1 == (idx - row * _LANES)
        sx1 = jnp.max(jnp.where(lmask, x1s[pl.ds(row, 1), :], neg_inf))
        sy1 = jnp.max(jnp.where(lmask, y1s[pl.ds(row, 1), :], neg_inf))
        sx2 = jnp.max(jnp.where(lmask, x2s[pl.ds(row, 1), :], neg_inf))
        sy2 = jnp.max(jnp.where(lmask, y2s[pl.ds(row, 1), :], neg_inf))
        ssc = jnp.max(jnp.where(lmask, s_ref[pl.ds(row, 1), :], neg_inf))

        is_deg = m == f32(_NEG)
        dx1 = jnp.where(i == 0, sx1, dx1)
        dy1 = jnp.where(i == 0, sy1, dy1)
        dx2 = jnp.where(i == 0, sx2, dx2)
        dy2 = jnp.where(i == 0, sy2, dy2)
        ds = jnp.where(i == 0, ssc, ds)
        sx1 = jnp.where(is_deg, dx1, sx1)
        sy1 = jnp.where(is_deg, dy1, sy1)
        sx2 = jnp.where(is_deg, dx2, sx2)
        sy2 = jnp.where(is_deg, dy2, sy2)
        ssc = jnp.where(is_deg, ds, ssc)

        xx1 = jnp.maximum(sx1, x1s[:])
        yy1 = jnp.maximum(sy1, y1s[:])
        xx2 = jnp.minimum(sx2, x2s[:])
        yy2 = jnp.minimum(sy2, y2s[:])
        w = jnp.maximum(xx2 - xx1, f32(0.0))
        h = jnp.maximum(yy2 - yy1, f32(0.0))
        inter = w * h
        sel_area = (sx2 - sx1) * (sy2 - sy1)
        iou = inter / (areas_s[:] + sel_area - inter + f32(1e-9))
        s = jnp.where((iou > f32(_IOU)) | mask2, f32(_NEG), s)

        out = (jnp.where(lane1 == 0, sx1, f32(0.0))
               + jnp.where(lane1 == 1, sy1, f32(0.0))
               + jnp.where(lane1 == 2, sx2, f32(0.0))
               + jnp.where(lane1 == 3, sy2, f32(0.0))
               + jnp.where(lane1 == 4, ssc, f32(0.0)))
        out_ref[pl.ds(i, 1), :] = out
        return (s, dx1, dy1, dx2, dy2, ds)

    zero = f32(0.0)
    lax.fori_loop(0, _NOUT, step, (s0, zero, zero, zero, zero, zero))


@jax.jit
def kernel(boxes, scores):
    pad = _P - _N
    comps = [
        jnp.pad(boxes[:, i], (0, pad)).reshape(_ROWS, _LANES) for i in range(4)
    ]
    s = jnp.pad(scores, (0, pad)).reshape(_ROWS, _LANES)
    scratch = [pltpu.VMEM((_ROWS, _LANES), jnp.float32)] * 5
    out = pl.pallas_call(
        _nms_body,
        out_shape=jax.ShapeDtypeStruct((_NOUT, _LANES), jnp.float32),
        scratch_shapes=scratch,
    )(*comps, s)
    return out[:, :5]


# speedup vs baseline: 2.7743x; 1.0679x over previous
import functools

import jax
import jax.numpy as jnp
from jax import lax
from jax.experimental import pallas as pl
from jax.experimental.pallas import tpu as pltpu

_N = 20000
_K = 6000
_NOUT = 300
_IOU = 0.7
_SCALE = 1000.0
_ROWS = 160
_LANES = 128
_P = _ROWS * _LANES
_C = 6144
_CROWS = 48
_NEG = -1e9


def _nms_body(c0_ref, c1_ref, c2_ref, c3_ref, s_ref, out_ref,
              x1s, y1s, x2s, y2s, scs, areas_s):
    f32 = jnp.float32
    i32 = jnp.int32
    imin = jnp.int32(-2147483648)

    row_i = lax.broadcasted_iota(i32, (_ROWS, _LANES), 0)
    lane_i = lax.broadcasted_iota(i32, (_ROWS, _LANES), 1)
    flat_i = row_i * _LANES + lane_i
    flat_c = lax.broadcasted_iota(i32, (_CROWS, _LANES), 0) * _LANES \
        + lax.broadcasted_iota(i32, (_CROWS, _LANES), 1)
    lane1 = lax.broadcasted_iota(i32, (1, _LANES), 1)
    valid = flat_i < _N

    b0 = c0_ref[:] * _SCALE
    b1 = c1_ref[:] * _SCALE
    b2 = c2_ref[:] * _SCALE
    b3 = c3_ref[:] * _SCALE
    x1 = jnp.minimum(b0, b2)
    x2 = jnp.maximum(b0, b2)
    y1 = jnp.minimum(b1, b3)
    y2 = jnp.maximum(b1, b3)
    scores = s_ref[:]

    bits = lax.bitcast_convert_type(scores, i32)
    akey = bits ^ (lax.shift_right_arithmetic(bits, 31) & jnp.int32(0x7FFFFFFF))
    akey = jnp.where(valid, akey, imin)
    inv = _P - flat_i

    Tf = imin
    Ti = jnp.int32(0)
    for b in range(31, -1, -1):
        trial = (Tf ^ imin) if b == 31 else (Tf | jnp.int32(1 << b))
        cnt = jnp.sum((akey >= trial).astype(i32))
        Tf = jnp.where(cnt >= _K, trial, Tf)
    for b in range(14, -1, -1):
        trial = Ti | jnp.int32(1 << b)
        cond = (akey > Tf) | ((akey == Tf) & (inv >= trial))
        cnt = jnp.sum(cond.astype(i32))
        Ti = jnp.where(cnt >= _K, trial, Ti)
    in_set = (akey > Tf) | ((akey == Tf) & (inv >= Ti))

    bf16 = jnp.bfloat16
    mask_bf = in_set.astype(bf16)
    up = (lax.broadcasted_iota(i32, (_LANES, _LANES), 0)
          < lax.broadcasted_iota(i32, (_LANES, _LANES), 1)).astype(bf16)
    lane_excl = lax.dot_general(mask_bf, up, (((1,), (0,)), ((), ())),
                                preferred_element_type=jnp.float32)
    rowsum = jnp.sum(in_set.astype(f32), axis=1, keepdims=True)
    lo = (lax.broadcasted_iota(i32, (_ROWS, _ROWS), 0)
          > lax.broadcasted_iota(i32, (_ROWS, _ROWS), 1)).astype(bf16)
    row_excl = lax.dot_general(lo, rowsum.astype(bf16), (((1,), (0,)), ((), ())),
                               preferred_element_type=jnp.float32)
    rank = (row_excl + lane_excl).astype(i32)

    d = jnp.where(in_set, flat_i - rank, 0)

    def shift_down(v, k):
        r, l = divmod(k, _LANES)
        if l:
            a = jnp.concatenate([v[:, l:], v[:, :l]], axis=1)
            b_ = jnp.concatenate([a[1:, :], a[:1, :]], axis=0)
            v = jnp.where(lane_i < _LANES - l, a, b_)
        if r:
            v = jnp.concatenate([v[r:, :], v[:r, :]], axis=0)
        return v

    for b in range(14):
        k = 1 << b
        dr = shift_down(d, k)
        mv = (lax.shift_right_logical(dr, b) & 1) == 1
        x1 = jnp.where(mv, shift_down(x1, k), x1)
        y1 = jnp.where(mv, shift_down(y1, k), y1)
        x2 = jnp.where(mv, shift_down(x2, k), x2)
        y2 = jnp.where(mv, shift_down(y2, k), y2)
        scores = jnp.where(mv, shift_down(scores, k), scores)
        d = jnp.where(mv, dr, d)

    x1c = x1[:_CROWS]
    y1c = y1[:_CROWS]
    x2c = x2[:_CROWS]
    y2c = y2[:_CROWS]
    scc = scores[:_CROWS]
    x1s[:] = x1c
    y1s[:] = y1c
    x2s[:] = x2c
    y2s[:] = y2c
    scs[:] = scc
    areas = (x2c - x1c) * (y2c - y1c)
    areas_s[:] = areas
    s0 = jnp.where(flat_c < _K, scc, f32(_NEG))
    neg_inf = f32(-jnp.inf)

    def step(i, carry):
        s, dx1, dy1, dx2, dy2, ds = carry
        m = jnp.max(s)
        idx = jnp.min(jnp.where(s == m, flat_c, _C))
        mask2 = flat_c == idx
        row = idx // _LANES
        lmask = lane1 == (idx - row * _LANES)
        sx1 = jnp.max(jnp.where(lmask, x1s[pl.ds(row, 1), :], neg_inf))
        sy1 = jnp.max(jnp.where(lmask, y1s[pl.ds(row, 1), :], neg_inf))
        sx2 = jnp.max(jnp.where(lmask, x2s[pl.ds(row, 1), :], neg_inf))
        sy2 = jnp.max(jnp.where(lmask, y2s[pl.ds(row, 1), :], neg_inf))
        ssc = jnp.max(jnp.where(lmask, scs[pl.ds(row, 1), :], neg_inf))

        is_deg = m == f32(_NEG)
        dx1 = jnp.where(i == 0, sx1, dx1)
        dy1 = jnp.where(i == 0, sy1, dy1)
        dx2 = jnp.where(i == 0, sx2, dx2)
        dy2 = jnp.where(i == 0, sy2, dy2)
        ds = jnp.where(i == 0, ssc, ds)
        sx1 = jnp.where(is_deg, dx1, sx1)
        sy1 = jnp.where(is_deg, dy1, sy1)
        sx2 = jnp.where(is_deg, dx2, sx2)
        sy2 = jnp.where(is_deg, dy2, sy2)
        ssc = jnp.where(is_deg, ds, ssc)

        xx1 = jnp.maximum(sx1, x1s[:])
        yy1 = jnp.maximum(sy1, y1s[:])
        xx2 = jnp.minimum(sx2, x2s[:])
        yy2 = jnp.minimum(sy2, y2s[:])
        w = jnp.maximum(xx2 - xx1, f32(0.0))
        h = jnp.maximum(yy2 - yy1, f32(0.0))
        inter = w * h
        sel_area = (sx2 - sx1) * (sy2 - sy1)
        iou = inter / (areas_s[:] + sel_area - inter + f32(1e-9))
        s = jnp.where((iou > f32(_IOU)) | mask2, f32(_NEG), s)

        out = (jnp.where(lane1 == 0, sx1, f32(0.0))
               + jnp.where(lane1 == 1, sy1, f32(0.0))
               + jnp.where(lane1 == 2, sx2, f32(0.0))
               + jnp.where(lane1 == 3, sy2, f32(0.0))
               + jnp.where(lane1 == 4, ssc, f32(0.0)))
        out_ref[pl.ds(i, 1), :] = out
        return (s, dx1, dy1, dx2, dy2, ds)

    zero = f32(0.0)
    lax.fori_loop(0, _NOUT, step, (s0, zero, zero, zero, zero, zero))


@jax.jit
def kernel(boxes, scores):
    pad = _P - _N
    comps = [
        jnp.pad(boxes[:, i], (0, pad)).reshape(_ROWS, _LANES) for i in range(4)
    ]
    s = jnp.pad(scores, (0, pad)).reshape(_ROWS, _LANES)
    scratch = [pltpu.VMEM((_CROWS, _LANES), jnp.float32)] * 6
    out = pl.pallas_call(
        _nms_body,
        out_shape=jax.ShapeDtypeStruct((_NOUT, _LANES), jnp.float32),
        scratch_shapes=scratch,
    )(*comps, s)
    return out[:, :5]
